# Initial kernel scaffold; baseline (speedup 1.0000x reference)
import math

import jax
import jax.numpy as jnp
from jax import lax
from jax.experimental import pallas as pl
from jax.experimental.pallas import tpu as pltpu
from jax.experimental.pallas import tpu_sc as plsc

N_NODES = 10000
N_EDGES = 320000
D = 128

NC = 2
NS = 16
NW = NC * NS
E_PER_TILE = N_EDGES // NW
CHUNK = 128
N_FULL = E_PER_TILE // CHUNK
REM = E_PER_TILE - N_FULL * CHUNK
ROWS_PER_TILE = N_NODES // NS
ZROWS = 125


def _sc_body(x_hbm, src_hbm, dst_hbm, he_hbm,
             src_v, dst_v, rows_v, srcr_v, dstr_v, rowsr_v, zbuf, he_sh, sem):
    cid = lax.axis_index("c")
    sid = lax.axis_index("s")
    wid = cid * NS + sid

    zeros16 = jnp.zeros((16,), jnp.float32)

    def _zb(i, carry):
        r = i // (D // 16)
        c = (i % (D // 16)) * 16
        zbuf[r, pl.ds(c, 16)] = zeros16
        return carry

    lax.fori_loop(0, ZROWS * (D // 16), _zb, 0)
    for part in range(ROWS_PER_TILE // ZROWS):
        pltpu.sync_copy(
            zbuf, he_sh.at[pl.ds(sid * ROWS_PER_TILE + part * ZROWS, ZROWS)])
    plsc.subcore_barrier()

    ebase = wid * E_PER_TILE

    def _chunk(j, carry):
        base = pl.multiple_of(ebase + j * CHUNK, 8)
        pltpu.sync_copy(src_hbm.at[pl.ds(base, CHUNK)], src_v)
        pltpu.sync_copy(dst_hbm.at[pl.ds(base, CHUNK)], dst_v)
        pltpu.async_copy(x_hbm.at[src_v], rows_v, sem).wait()
        pltpu.sync_copy(rows_v, he_sh.at[dst_v], add=True)
        return carry

    lax.fori_loop(0, N_FULL, _chunk, 0)

    rbase = ebase + N_FULL * CHUNK
    pltpu.sync_copy(src_hbm.at[pl.ds(rbase, REM)], srcr_v)
    pltpu.sync_copy(dst_hbm.at[pl.ds(rbase, REM)], dstr_v)
    pltpu.async_copy(x_hbm.at[srcr_v], rowsr_v, sem).wait()
    pltpu.sync_copy(rowsr_v, he_sh.at[dstr_v], add=True)

    plsc.subcore_barrier()
    pltpu.sync_copy(
        he_sh.at[pl.ds(sid * ROWS_PER_TILE, ROWS_PER_TILE)],
        he_hbm.at[cid, pl.ds(sid * ROWS_PER_TILE, ROWS_PER_TILE)])


_sc_segsum = pl.kernel(
    _sc_body,
    out_type=jax.ShapeDtypeStruct((NC, N_NODES, D), jnp.float32),
    mesh=plsc.VectorSubcoreMesh(core_axis_name="c", subcore_axis_name="s"),
    scratch_types=[
        pltpu.VMEM((CHUNK,), jnp.int32),
        pltpu.VMEM((CHUNK,), jnp.int32),
        pltpu.VMEM((CHUNK, D), jnp.float32),
        pltpu.VMEM((REM,), jnp.int32),
        pltpu.VMEM((REM,), jnp.int32),
        pltpu.VMEM((REM, D), jnp.float32),
        pltpu.VMEM((ZROWS, D), jnp.float32),
        pltpu.VMEM_SHARED((N_NODES, D), jnp.float32),
        pltpu.SemaphoreType.DMA,
    ],
)


def _tc_body(x_ref, he_ref, w_ref, b_ref, o_ref):
    acc = x_ref[...] + math.pi * (he_ref[0] + he_ref[1])
    o_ref[...] = lax.dot_general(
        acc, w_ref[...], (((1,), (1,)), ((), ())),
        preferred_element_type=jnp.float32) + b_ref[...]


def _tc_linear(x, he, W, b2d):
    blk = 1000
    grid = N_NODES // blk
    return pl.pallas_call(
        _tc_body,
        grid=(grid,),
        in_specs=[
            pl.BlockSpec((blk, D), lambda i: (i, 0)),
            pl.BlockSpec((NC, blk, D), lambda i: (0, i, 0)),
            pl.BlockSpec((D, D), lambda i: (0, 0)),
            pl.BlockSpec((1, D), lambda i: (0, 0)),
        ],
        out_specs=pl.BlockSpec((blk, D), lambda i: (i, 0)),
        out_shape=jax.ShapeDtypeStruct((N_NODES, D), jnp.float32),
    )(x, he, W, b2d)


def kernel(x, edge_index, W, b):
    src = edge_index[0]
    dst = edge_index[1]
    he = _sc_segsum(x, src, dst)
    return _tc_linear(x, he, W, b.reshape(1, D))


# speedup vs baseline: 6.7141x; 6.7141x over previous
import math

import jax
import jax.numpy as jnp
from jax import lax
from jax.experimental import pallas as pl
from jax.experimental.pallas import tpu as pltpu
from jax.experimental.pallas import tpu_sc as plsc

N_NODES = 10000
N_EDGES = 320000
D = 128

NC = 2
NS = 16
NW = NC * NS
E_PER_TILE = N_EDGES // NW
CHUNK = 128
N_FULL = E_PER_TILE // CHUNK
REM = E_PER_TILE - N_FULL * CHUNK
N_PAD = 10240
ROWS_PER_TILE = N_PAD // NS
ZROWS = 128


def _sc_body(x_hbm, src_hbm, dst_hbm, he_hbm,
             src_v, dst_v, rows_v, srcr_v, dstr_v, rowsr_v, zbuf, he_sh, sem):
    cid = lax.axis_index("c")
    sid = lax.axis_index("s")
    wid = cid * NS + sid

    zeros16 = jnp.zeros((16,), jnp.float32)

    def _zb(i, carry):
        r = i // (D // 16)
        c = (i % (D // 16)) * 16
        zbuf[r, pl.ds(c, 16)] = zeros16
        return carry

    lax.fori_loop(0, ZROWS * (D // 16), _zb, 0)
    for part in range(ROWS_PER_TILE // ZROWS):
        pltpu.sync_copy(
            zbuf, he_sh.at[pl.ds(sid * ROWS_PER_TILE + part * ZROWS, ZROWS)])
    plsc.subcore_barrier()

    ebase = wid * E_PER_TILE

    def _chunk(j, carry):
        base = pl.multiple_of(ebase + j * CHUNK, 8)
        pltpu.sync_copy(src_hbm.at[pl.ds(base, CHUNK)], src_v)
        pltpu.sync_copy(dst_hbm.at[pl.ds(base, CHUNK)], dst_v)
        pltpu.async_copy(x_hbm.at[src_v], rows_v, sem).wait()
        pltpu.sync_copy(rows_v, he_sh.at[dst_v], add=True)
        return carry

    lax.fori_loop(0, N_FULL, _chunk, 0)

    rbase = ebase + N_FULL * CHUNK
    pltpu.sync_copy(src_hbm.at[pl.ds(rbase, REM)], srcr_v)
    pltpu.sync_copy(dst_hbm.at[pl.ds(rbase, REM)], dstr_v)
    pltpu.async_copy(x_hbm.at[srcr_v], rowsr_v, sem).wait()
    pltpu.sync_copy(rowsr_v, he_sh.at[dstr_v], add=True)

    plsc.subcore_barrier()
    pltpu.sync_copy(
        he_sh.at[pl.ds(sid * ROWS_PER_TILE, ROWS_PER_TILE)],
        he_hbm.at[cid, pl.ds(sid * ROWS_PER_TILE, ROWS_PER_TILE)])


_sc_segsum = pl.kernel(
    _sc_body,
    out_type=jax.ShapeDtypeStruct((NC, N_PAD, D), jnp.float32),
    mesh=plsc.VectorSubcoreMesh(core_axis_name="c", subcore_axis_name="s"),
    scratch_types=[
        pltpu.VMEM((CHUNK,), jnp.int32),
        pltpu.VMEM((CHUNK,), jnp.int32),
        pltpu.VMEM((CHUNK, D), jnp.float32),
        pltpu.VMEM((REM,), jnp.int32),
        pltpu.VMEM((REM,), jnp.int32),
        pltpu.VMEM((REM, D), jnp.float32),
        pltpu.VMEM((ZROWS, D), jnp.float32),
        pltpu.VMEM_SHARED((N_PAD, D), jnp.float32),
        pltpu.SemaphoreType.DMA,
    ],
)


def _tc_body(x_ref, he_ref, w_ref, b_ref, o_ref):
    acc = x_ref[...] + math.pi * (he_ref[0] + he_ref[1])
    o_ref[...] = lax.dot_general(
        acc, w_ref[...], (((1,), (1,)), ((), ())),
        preferred_element_type=jnp.float32) + b_ref[...]


def _tc_linear(x, he, W, b2d):
    blk = 1000
    grid = N_NODES // blk
    return pl.pallas_call(
        _tc_body,
        grid=(grid,),
        in_specs=[
            pl.BlockSpec((blk, D), lambda i: (i, 0)),
            pl.BlockSpec((NC, blk, D), lambda i: (0, i, 0)),
            pl.BlockSpec((D, D), lambda i: (0, 0)),
            pl.BlockSpec((1, D), lambda i: (0, 0)),
        ],
        out_specs=pl.BlockSpec((blk, D), lambda i: (i, 0)),
        out_shape=jax.ShapeDtypeStruct((N_NODES, D), jnp.float32),
    )(x, he, W, b2d)


def kernel(x, edge_index, W, b):
    src = edge_index[0]
    dst = edge_index[1]
    he = _sc_segsum(x, src, dst)
    return _tc_linear(x, he, W, b.reshape(1, D))


# speedup vs baseline: 11.5973x; 1.7273x over previous
import math

import jax
import jax.numpy as jnp
from jax import lax
from jax.experimental import pallas as pl
from jax.experimental.pallas import tpu as pltpu
from jax.experimental.pallas import tpu_sc as plsc

N_NODES = 10000
N_EDGES = 320000
D = 128

NC = 2
NS = 16
NW = NC * NS
E_PER_TILE = N_EDGES // NW
CHUNK = 128
N_FULL = E_PER_TILE // CHUNK
REM = E_PER_TILE - N_FULL * CHUNK
N_PAD = 10240
ROWS_PER_TILE = N_PAD // NS


def _sc_body(x_hbm, src_hbm, dst_hbm, zeros_hbm, he_hbm,
             rows, sb0, sb1, db0, db1,
             srcr_v, dstr_v, rowsr_v, he_sh,
             gsem0, gsem1, isem0, isem1, sem):
    cid = lax.axis_index("c")
    sid = lax.axis_index("s")
    wid = cid * NS + sid
    gsems = (gsem0, gsem1)
    isems = (isem0, isem1)
    sbufs = (sb0, sb1)
    dbufs = (db0, db1)

    pltpu.sync_copy(zeros_hbm, he_sh.at[pl.ds(sid * ROWS_PER_TILE, ROWS_PER_TILE)])
    plsc.subcore_barrier()

    ebase = wid * E_PER_TILE

    def rows_at(b):
        return rows.at[pl.ds(b * CHUNK, CHUNK)]

    def eslice(j):
        return pl.ds(pl.multiple_of(ebase + j * CHUNK, 8), CHUNK)

    def start_idx(j, b):
        pltpu.async_copy(src_hbm.at[eslice(j)], sbufs[b], isems[b])
        pltpu.async_copy(dst_hbm.at[eslice(j)], dbufs[b], isems[b])

    def wait_idx(j, b):
        pltpu.make_async_copy(src_hbm.at[eslice(j)], sbufs[b], isems[b]).wait()
        pltpu.make_async_copy(dst_hbm.at[eslice(j)], dbufs[b], isems[b]).wait()

    def fire_gather(j, b):
        pltpu.async_copy(x_hbm.at[sbufs[b]], rows_at(b), gsems[b])

    def drain_gather(b):
        pltpu.make_async_copy(x_hbm.at[sbufs[b]], rows_at(b), gsems[b]).wait()

    def scatter(b):
        pltpu.sync_copy(rows_at(b), he_sh.at[dbufs[b]], add=True)

    start_idx(0, 0)
    wait_idx(0, 0)
    fire_gather(0, 0)
    start_idx(1, 1)

    def _body(j, b):
        bo = b ^ 1
        wait_idx(j + 1, bo)
        fire_gather(j + 1, bo)
        drain_gather(b)
        scatter(b)
        start_idx(j + 2, b)

    def _pipe(t, carry):
        _body(2 * t, 0)
        _body(2 * t + 1, 1)
        return carry

    lax.fori_loop(0, (N_FULL - 2) // 2, _pipe, 0)

    wait_idx(N_FULL - 1, 1)
    fire_gather(N_FULL - 1, 1)
    drain_gather(0)
    scatter(0)
    drain_gather(1)
    scatter(1)

    rbase = ebase + N_FULL * CHUNK
    pltpu.sync_copy(src_hbm.at[pl.ds(rbase, REM)], srcr_v)
    pltpu.sync_copy(dst_hbm.at[pl.ds(rbase, REM)], dstr_v)
    pltpu.async_copy(x_hbm.at[srcr_v], rowsr_v, sem).wait()
    pltpu.sync_copy(rowsr_v, he_sh.at[dstr_v], add=True)

    plsc.subcore_barrier()
    pltpu.sync_copy(
        he_sh.at[pl.ds(sid * ROWS_PER_TILE, ROWS_PER_TILE)],
        he_hbm.at[cid, pl.ds(sid * ROWS_PER_TILE, ROWS_PER_TILE)])


_sc_segsum = pl.kernel(
    _sc_body,
    out_type=jax.ShapeDtypeStruct((NC, N_PAD, D), jnp.float32),
    mesh=plsc.VectorSubcoreMesh(core_axis_name="c", subcore_axis_name="s"),
    scratch_types=[
        pltpu.VMEM((2 * CHUNK, D), jnp.float32),
        pltpu.VMEM((CHUNK,), jnp.int32),
        pltpu.VMEM((CHUNK,), jnp.int32),
        pltpu.VMEM((CHUNK,), jnp.int32),
        pltpu.VMEM((CHUNK,), jnp.int32),
        pltpu.VMEM((REM,), jnp.int32),
        pltpu.VMEM((REM,), jnp.int32),
        pltpu.VMEM((REM, D), jnp.float32),
        pltpu.VMEM_SHARED((N_PAD, D), jnp.float32),
        pltpu.SemaphoreType.DMA,
        pltpu.SemaphoreType.DMA,
        pltpu.SemaphoreType.DMA,
        pltpu.SemaphoreType.DMA,
        pltpu.SemaphoreType.DMA,
    ],
)


def _tc_body(x_ref, he_ref, w_ref, b_ref, o_ref):
    acc = x_ref[...] + math.pi * (he_ref[0] + he_ref[1])
    o_ref[...] = lax.dot_general(
        acc, w_ref[...], (((1,), (1,)), ((), ())),
        preferred_element_type=jnp.float32) + b_ref[...]


def _tc_linear(x, he, W, b2d):
    blk = 1000
    grid = N_NODES // blk
    return pl.pallas_call(
        _tc_body,
        grid=(grid,),
        in_specs=[
            pl.BlockSpec((blk, D), lambda i: (i, 0)),
            pl.BlockSpec((NC, blk, D), lambda i: (0, i, 0)),
            pl.BlockSpec((D, D), lambda i: (0, 0)),
            pl.BlockSpec((1, D), lambda i: (0, 0)),
        ],
        out_specs=pl.BlockSpec((blk, D), lambda i: (i, 0)),
        out_shape=jax.ShapeDtypeStruct((N_NODES, D), jnp.float32),
    )(x, he, W, b2d)


def kernel(x, edge_index, W, b):
    src = edge_index[0]
    dst = edge_index[1]
    zeros = jnp.zeros((ROWS_PER_TILE, D), jnp.float32)
    he = _sc_segsum(x, src, dst, zeros)
    return _tc_linear(x, he, W, b.reshape(1, D))


# speedup vs baseline: 11.6171x; 1.0017x over previous
import math

import jax
import jax.numpy as jnp
from jax import lax
from jax.experimental import pallas as pl
from jax.experimental.pallas import tpu as pltpu
from jax.experimental.pallas import tpu_sc as plsc

N_NODES = 10000
N_EDGES = 320000
D = 128

NC = 2
NS = 16
NW = NC * NS
E_PER_TILE = N_EDGES // NW
CHUNK = 128
N_FULL = E_PER_TILE // CHUNK
REM = E_PER_TILE - N_FULL * CHUNK
N_PAD = 10240
ROWS_PER_TILE = N_PAD // NS


def _sc_body(x_hbm, src_hbm, dst_hbm, zeros_hbm, he_hbm,
             rows, sb0, sb1, db0, db1,
             srcr_v, dstr_v, rowsr_v, he_sh,
             gsem0, gsem1, isem0, isem1, sem):
    cid = lax.axis_index("c")
    sid = lax.axis_index("s")
    wid = cid * NS + sid
    gsems = (gsem0, gsem1)
    isems = (isem0, isem1)
    sbufs = (sb0, sb1)
    dbufs = (db0, db1)

    with jax.named_scope("zero_acc"):
        pltpu.sync_copy(zeros_hbm, he_sh.at[pl.ds(sid * ROWS_PER_TILE, ROWS_PER_TILE)])
        plsc.subcore_barrier()

    ebase = wid * E_PER_TILE

    def rows_at(b):
        return rows.at[pl.ds(b * CHUNK, CHUNK)]

    def eslice(j):
        return pl.ds(pl.multiple_of(ebase + j * CHUNK, 8), CHUNK)

    def start_idx(j, b):
        pltpu.async_copy(src_hbm.at[eslice(j)], sbufs[b], isems[b])
        pltpu.async_copy(dst_hbm.at[eslice(j)], dbufs[b], isems[b])

    def wait_idx(j, b):
        pltpu.make_async_copy(src_hbm.at[eslice(j)], sbufs[b], isems[b]).wait()
        pltpu.make_async_copy(dst_hbm.at[eslice(j)], dbufs[b], isems[b]).wait()

    def fire_gather(j, b):
        pltpu.async_copy(x_hbm.at[sbufs[b]], rows_at(b), gsems[b])

    def drain_gather(b):
        pltpu.make_async_copy(x_hbm.at[sbufs[b]], rows_at(b), gsems[b]).wait()

    def scatter(b):
        pltpu.sync_copy(rows_at(b), he_sh.at[dbufs[b]], add=True)

    scope = jax.named_scope("edge_pipeline")
    scope.__enter__()
    start_idx(0, 0)
    wait_idx(0, 0)
    fire_gather(0, 0)
    start_idx(1, 1)

    def _body(j, b):
        bo = b ^ 1
        wait_idx(j + 1, bo)
        fire_gather(j + 1, bo)
        drain_gather(b)
        scatter(b)
        start_idx(j + 2, b)

    def _pipe(t, carry):
        _body(2 * t, 0)
        _body(2 * t + 1, 1)
        return carry

    lax.fori_loop(0, (N_FULL - 2) // 2, _pipe, 0)

    wait_idx(N_FULL - 1, 1)
    fire_gather(N_FULL - 1, 1)
    drain_gather(0)
    scatter(0)
    drain_gather(1)
    scatter(1)

    rbase = ebase + N_FULL * CHUNK
    pltpu.sync_copy(src_hbm.at[pl.ds(rbase, REM)], srcr_v)
    pltpu.sync_copy(dst_hbm.at[pl.ds(rbase, REM)], dstr_v)
    pltpu.async_copy(x_hbm.at[srcr_v], rowsr_v, sem).wait()
    pltpu.sync_copy(rowsr_v, he_sh.at[dstr_v], add=True)
    scope.__exit__(None, None, None)

    with jax.named_scope("writeout"):
        plsc.subcore_barrier()
        pltpu.sync_copy(
            he_sh.at[pl.ds(sid * ROWS_PER_TILE, ROWS_PER_TILE)],
            he_hbm.at[cid, pl.ds(sid * ROWS_PER_TILE, ROWS_PER_TILE)])


_sc_segsum = pl.kernel(
    _sc_body,
    out_type=jax.ShapeDtypeStruct((NC, N_PAD, D), jnp.float32),
    mesh=plsc.VectorSubcoreMesh(core_axis_name="c", subcore_axis_name="s"),
    scratch_types=[
        pltpu.VMEM((2 * CHUNK, D), jnp.float32),
        pltpu.VMEM((CHUNK,), jnp.int32),
        pltpu.VMEM((CHUNK,), jnp.int32),
        pltpu.VMEM((CHUNK,), jnp.int32),
        pltpu.VMEM((CHUNK,), jnp.int32),
        pltpu.VMEM((REM,), jnp.int32),
        pltpu.VMEM((REM,), jnp.int32),
        pltpu.VMEM((REM, D), jnp.float32),
        pltpu.VMEM_SHARED((N_PAD, D), jnp.float32),
        pltpu.SemaphoreType.DMA,
        pltpu.SemaphoreType.DMA,
        pltpu.SemaphoreType.DMA,
        pltpu.SemaphoreType.DMA,
        pltpu.SemaphoreType.DMA,
    ],
)


def _tc_body(x_ref, he_ref, w_ref, b_ref, o_ref):
    acc = x_ref[...] + math.pi * (he_ref[0] + he_ref[1])
    o_ref[...] = lax.dot_general(
        acc, w_ref[...], (((1,), (1,)), ((), ())),
        preferred_element_type=jnp.float32) + b_ref[...]


def _tc_linear(x, he, W, b2d):
    blk = 1000
    grid = N_NODES // blk
    return pl.pallas_call(
        _tc_body,
        grid=(grid,),
        in_specs=[
            pl.BlockSpec((blk, D), lambda i: (i, 0)),
            pl.BlockSpec((NC, blk, D), lambda i: (0, i, 0)),
            pl.BlockSpec((D, D), lambda i: (0, 0)),
            pl.BlockSpec((1, D), lambda i: (0, 0)),
        ],
        out_specs=pl.BlockSpec((blk, D), lambda i: (i, 0)),
        out_shape=jax.ShapeDtypeStruct((N_NODES, D), jnp.float32),
    )(x, he, W, b2d)


def kernel(x, edge_index, W, b):
    src = edge_index[0]
    dst = edge_index[1]
    zeros = jnp.zeros((ROWS_PER_TILE, D), jnp.float32)
    he = _sc_segsum(x, src, dst, zeros)
    return _tc_linear(x, he, W, b.reshape(1, D))


# speedup vs baseline: 12.7846x; 1.1005x over previous
import math

import jax
import jax.numpy as jnp
from jax import lax
from jax.experimental import pallas as pl
from jax.experimental.pallas import tpu as pltpu
from jax.experimental.pallas import tpu_sc as plsc

N_NODES = 10000
N_EDGES = 320000
D = 128

NC = 2
NS = 16
NW = NC * NS
E_PER_TILE = N_EDGES // NW
CHUNK = 128
N_FULL = E_PER_TILE // CHUNK
REM = E_PER_TILE - N_FULL * CHUNK
N_PAD = 10240
ROWS_PER_TILE = N_PAD // NS


def _sc_body(x_hbm, src_hbm, dst_hbm, zeros_hbm, he_hbm,
             rows, sb0, sb1, db0, db1, cb0, cb1,
             srcr_v, dstr_v, rowsr_v, he_sh,
             gsem0, gsem1, isem0, isem1, ssem0, ssem1, sem):
    cid = lax.axis_index("c")
    sid = lax.axis_index("s")
    wid = cid * NS + sid
    gsems = (gsem0, gsem1)
    isems = (isem0, isem1)
    ssems = (ssem0, ssem1)
    sbufs = (sb0, sb1)
    dbufs = (db0, db1)
    cbufs = (cb0, cb1)

    with jax.named_scope("zero_acc"):
        pltpu.sync_copy(zeros_hbm, he_sh.at[pl.ds(sid * ROWS_PER_TILE, ROWS_PER_TILE)])
        plsc.subcore_barrier()

    ebase = wid * E_PER_TILE

    def rows_at(b):
        return rows.at[pl.ds(b * CHUNK, CHUNK)]

    def eslice(j):
        return pl.ds(pl.multiple_of(ebase + j * CHUNK, 8), CHUNK)

    def start_idx(j, b):
        pltpu.async_copy(src_hbm.at[eslice(j)], sbufs[b], isems[b])
        pltpu.async_copy(dst_hbm.at[eslice(j)], dbufs[b], isems[b])

    def wait_idx(j, b):
        pltpu.make_async_copy(src_hbm.at[eslice(j)], sbufs[b], isems[b]).wait()
        pltpu.make_async_copy(dst_hbm.at[eslice(j)], dbufs[b], isems[b]).wait()

    def fire_gather(j, b):
        pltpu.async_copy(x_hbm.at[sbufs[b]], rows_at(b), gsems[b])

    def drain_gather(b):
        pltpu.make_async_copy(x_hbm.at[sbufs[b]], rows_at(b), gsems[b]).wait()

    def fire_scatter(b):
        for i in range(CHUNK // 16):
            cbufs[b][pl.ds(i * 16, 16)] = dbufs[b][pl.ds(i * 16, 16)]
        pltpu.async_copy(rows_at(b), he_sh.at[cbufs[b]], ssems[b], add=True)

    def drain_scatter(b):
        pltpu.make_async_copy(rows_at(b), he_sh.at[cbufs[b]], ssems[b]).wait()

    scope = jax.named_scope("edge_pipeline")
    scope.__enter__()
    start_idx(0, 0)
    wait_idx(0, 0)
    fire_gather(0, 0)
    start_idx(1, 1)
    wait_idx(1, 1)
    fire_gather(1, 1)
    drain_gather(0)
    fire_scatter(0)
    start_idx(2, 0)

    def _body(j, b, prefetch=True):
        bo = b ^ 1
        drain_scatter(bo)
        wait_idx(j + 1, bo)
        fire_gather(j + 1, bo)
        drain_gather(b)
        fire_scatter(b)
        if prefetch:
            start_idx(j + 2, b)

    def _pipe(t, carry):
        _body(2 * t + 1, 1)
        _body(2 * t + 2, 0)
        return carry

    lax.fori_loop(0, (N_FULL - 4) // 2, _pipe, 0)

    _body(N_FULL - 3, 1)
    _body(N_FULL - 2, 0, prefetch=False)
    drain_scatter(0)
    drain_gather(1)
    fire_scatter(1)
    drain_scatter(1)

    rbase = ebase + N_FULL * CHUNK
    pltpu.sync_copy(src_hbm.at[pl.ds(rbase, REM)], srcr_v)
    pltpu.sync_copy(dst_hbm.at[pl.ds(rbase, REM)], dstr_v)
    pltpu.async_copy(x_hbm.at[srcr_v], rowsr_v, sem).wait()
    pltpu.sync_copy(rowsr_v, he_sh.at[dstr_v], add=True)
    scope.__exit__(None, None, None)

    with jax.named_scope("writeout"):
        plsc.subcore_barrier()
        pltpu.sync_copy(
            he_sh.at[pl.ds(sid * ROWS_PER_TILE, ROWS_PER_TILE)],
            he_hbm.at[cid, pl.ds(sid * ROWS_PER_TILE, ROWS_PER_TILE)])


_sc_segsum = pl.kernel(
    _sc_body,
    out_type=jax.ShapeDtypeStruct((NC, N_PAD, D), jnp.float32),
    mesh=plsc.VectorSubcoreMesh(core_axis_name="c", subcore_axis_name="s"),
    scratch_types=[
        pltpu.VMEM((2 * CHUNK, D), jnp.float32),
        pltpu.VMEM((CHUNK,), jnp.int32),
        pltpu.VMEM((CHUNK,), jnp.int32),
        pltpu.VMEM((CHUNK,), jnp.int32),
        pltpu.VMEM((CHUNK,), jnp.int32),
        pltpu.VMEM((CHUNK,), jnp.int32),
        pltpu.VMEM((CHUNK,), jnp.int32),
        pltpu.VMEM((REM,), jnp.int32),
        pltpu.VMEM((REM,), jnp.int32),
        pltpu.VMEM((REM, D), jnp.float32),
        pltpu.VMEM_SHARED((N_PAD, D), jnp.float32),
        pltpu.SemaphoreType.DMA,
        pltpu.SemaphoreType.DMA,
        pltpu.SemaphoreType.DMA,
        pltpu.SemaphoreType.DMA,
        pltpu.SemaphoreType.DMA,
        pltpu.SemaphoreType.DMA,
        pltpu.SemaphoreType.DMA,
    ],
)


def _tc_body(x_ref, he_ref, w_ref, b_ref, o_ref):
    acc = x_ref[...] + math.pi * (he_ref[0] + he_ref[1])
    o_ref[...] = lax.dot_general(
        acc, w_ref[...], (((1,), (1,)), ((), ())),
        preferred_element_type=jnp.float32) + b_ref[...]


def _tc_linear(x, he, W, b2d):
    blk = 1000
    grid = N_NODES // blk
    return pl.pallas_call(
        _tc_body,
        grid=(grid,),
        in_specs=[
            pl.BlockSpec((blk, D), lambda i: (i, 0)),
            pl.BlockSpec((NC, blk, D), lambda i: (0, i, 0)),
            pl.BlockSpec((D, D), lambda i: (0, 0)),
            pl.BlockSpec((1, D), lambda i: (0, 0)),
        ],
        out_specs=pl.BlockSpec((blk, D), lambda i: (i, 0)),
        out_shape=jax.ShapeDtypeStruct((N_NODES, D), jnp.float32),
    )(x, he, W, b2d)


def kernel(x, edge_index, W, b):
    src = edge_index[0]
    dst = edge_index[1]
    zeros = jnp.zeros((ROWS_PER_TILE, D), jnp.float32)
    he = _sc_segsum(x, src, dst, zeros)
    return _tc_linear(x, he, W, b.reshape(1, D))


# speedup vs baseline: 13.3501x; 1.0442x over previous
import math

import jax
import jax.numpy as jnp
from jax import lax
from jax.experimental import pallas as pl
from jax.experimental.pallas import tpu as pltpu
from jax.experimental.pallas import tpu_sc as plsc

N_NODES = 10000
N_EDGES = 320000
D = 128

NC = 2
NS = 16
NW = NC * NS
E_PER_TILE = N_EDGES // NW
CHUNK = 80
NCH = E_PER_TILE // CHUNK
NSLOT = 4
N_PAD = 10240
ROWS_PER_TILE = N_PAD // NS


def _sc_body(x_hbm, src_hbm, dst_hbm, zeros_hbm, he_hbm,
             rows, sb0, sb1, sb2, sb3, db0, db1, db2, db3,
             cb0, cb1, cb2, cb3, he_sh,
             gsem0, gsem1, gsem2, gsem3,
             isem0, isem1, isem2, isem3,
             ssem0, ssem1, ssem2, ssem3):
    cid = lax.axis_index("c")
    sid = lax.axis_index("s")
    wid = cid * NS + sid
    gsems = (gsem0, gsem1, gsem2, gsem3)
    isems = (isem0, isem1, isem2, isem3)
    ssems = (ssem0, ssem1, ssem2, ssem3)
    sbufs = (sb0, sb1, sb2, sb3)
    dbufs = (db0, db1, db2, db3)
    cbufs = (cb0, cb1, cb2, cb3)

    with jax.named_scope("zero_acc"):
        pltpu.sync_copy(zeros_hbm, he_sh.at[pl.ds(sid * ROWS_PER_TILE, ROWS_PER_TILE)])
        plsc.subcore_barrier()

    ebase = wid * E_PER_TILE

    def rows_at(b):
        return rows.at[pl.ds(b * CHUNK, CHUNK)]

    def eslice(j):
        return pl.ds(pl.multiple_of(ebase + j * CHUNK, 8), CHUNK)

    def start_idx(j, b):
        pltpu.async_copy(src_hbm.at[eslice(j)], sbufs[b], isems[b])
        pltpu.async_copy(dst_hbm.at[eslice(j)], dbufs[b], isems[b])

    def wait_idx(j, b):
        pltpu.make_async_copy(src_hbm.at[eslice(j)], sbufs[b], isems[b]).wait()
        pltpu.make_async_copy(dst_hbm.at[eslice(j)], dbufs[b], isems[b]).wait()

    def fire_gather(b):
        pltpu.async_copy(x_hbm.at[sbufs[b]], rows_at(b), gsems[b])

    def drain_gather(b):
        pltpu.make_async_copy(x_hbm.at[sbufs[b]], rows_at(b), gsems[b]).wait()

    def fire_scatter(b):
        for i in range(CHUNK // 16):
            cbufs[b][pl.ds(i * 16, 16)] = dbufs[b][pl.ds(i * 16, 16)]
        pltpu.async_copy(rows_at(b), he_sh.at[cbufs[b]], ssems[b], add=True)

    def drain_scatter(b):
        pltpu.make_async_copy(rows_at(b), he_sh.at[cbufs[b]], ssems[b]).wait()

    scope = jax.named_scope("edge_pipeline")
    scope.__enter__()

    def body(j, b, steady=True, f=True):
        if steady:
            drain_scatter(b)
        wait_idx(j, b)
        fire_gather(b)
        if steady or b >= 2:
            drain_gather((b - 2) % 4)
            fire_scatter((b - 2) % 4)
        if f:
            start_idx(j + 2, (b + 2) % 4)

    start_idx(0, 0)
    start_idx(1, 1)
    for j in range(4):
        body(j, j, steady=False)

    def _pipe(t, carry):
        j0 = 4 + 4 * t
        for u in range(4):
            body(j0 + u, u)
        return carry

    lax.fori_loop(0, (NCH - 4 - 5) // 4, _pipe, 0)

    for j in range(NCH - 5, NCH):
        body(j, j % 4, f=(j + 2 < NCH))
    drain_gather((NCH - 2) % 4)
    fire_scatter((NCH - 2) % 4)
    drain_gather((NCH - 1) % 4)
    fire_scatter((NCH - 1) % 4)
    for j in range(NCH - 4, NCH):
        drain_scatter(j % 4)
    scope.__exit__(None, None, None)

    with jax.named_scope("writeout"):
        plsc.subcore_barrier()
        pltpu.sync_copy(
            he_sh.at[pl.ds(sid * ROWS_PER_TILE, ROWS_PER_TILE)],
            he_hbm.at[cid, pl.ds(sid * ROWS_PER_TILE, ROWS_PER_TILE)])


_sc_segsum = pl.kernel(
    _sc_body,
    out_type=jax.ShapeDtypeStruct((NC, N_PAD, D), jnp.float32),
    mesh=plsc.VectorSubcoreMesh(core_axis_name="c", subcore_axis_name="s"),
    scratch_types=(
        [pltpu.VMEM((NSLOT * CHUNK, D), jnp.float32)]
        + [pltpu.VMEM((CHUNK,), jnp.int32) for _ in range(12)]
        + [pltpu.VMEM_SHARED((N_PAD, D), jnp.float32)]
        + [pltpu.SemaphoreType.DMA for _ in range(12)]
    ),
)


def _tc_body(x_ref, he_ref, w_ref, b_ref, o_ref):
    acc = x_ref[...] + math.pi * (he_ref[0] + he_ref[1])
    o_ref[...] = lax.dot_general(
        acc, w_ref[...], (((1,), (1,)), ((), ())),
        preferred_element_type=jnp.float32) + b_ref[...]


def _tc_linear(x, he, W, b2d):
    blk = 1000
    grid = N_NODES // blk
    return pl.pallas_call(
        _tc_body,
        grid=(grid,),
        in_specs=[
            pl.BlockSpec((blk, D), lambda i: (i, 0)),
            pl.BlockSpec((NC, blk, D), lambda i: (0, i, 0)),
            pl.BlockSpec((D, D), lambda i: (0, 0)),
            pl.BlockSpec((1, D), lambda i: (0, 0)),
        ],
        out_specs=pl.BlockSpec((blk, D), lambda i: (i, 0)),
        out_shape=jax.ShapeDtypeStruct((N_NODES, D), jnp.float32),
    )(x, he, W, b2d)


def kernel(x, edge_index, W, b):
    src = edge_index[0]
    dst = edge_index[1]
    zeros = jnp.zeros((ROWS_PER_TILE, D), jnp.float32)
    he = _sc_segsum(x, src, dst, zeros)
    return _tc_linear(x, he, W, b.reshape(1, D))


# speedup vs baseline: 14.4338x; 1.0812x over previous
import math

import jax
import jax.numpy as jnp
from jax import lax
from jax.experimental import pallas as pl
from jax.experimental.pallas import tpu as pltpu
from jax.experimental.pallas import tpu_sc as plsc

N_NODES = 10000
N_EDGES = 320000
D = 128

NC = 2
NS = 16
NW = NC * NS
E_PER_TILE = N_EDGES // NW
CHUNK = 80
NCH = E_PER_TILE // CHUNK
NSLOT = 4
N_PAD = 10240
ROWS_PER_TILE = N_PAD // NS


def _sc_body(x_hbm, src_hbm, zeros_hbm, he_hbm,
             rows, sb0, sb1, sb2, sb3, db0, db1, db2, db3,
             cb0, cb1, cb2, cb3, he_sh,
             gsem0, gsem1, gsem2, gsem3,
             isem0, isem1, isem2, isem3,
             ssem0, ssem1, ssem2, ssem3):
    cid = lax.axis_index("c")
    sid = lax.axis_index("s")
    wid = cid * NS + sid
    gsems = (gsem0, gsem1, gsem2, gsem3)
    isems = (isem0, isem1, isem2, isem3)
    ssems = (ssem0, ssem1, ssem2, ssem3)
    sbufs = (sb0, sb1, sb2, sb3)
    dbufs = (db0, db1, db2, db3)
    cbufs = (cb0, cb1, cb2, cb3)

    with jax.named_scope("zero_acc"):
        pltpu.sync_copy(zeros_hbm, he_sh.at[pl.ds(sid * ROWS_PER_TILE, ROWS_PER_TILE)])
        plsc.subcore_barrier()

    ebase = wid * E_PER_TILE

    def rows_at(b):
        return rows.at[pl.ds(b * CHUNK, CHUNK)]

    def eslice(j):
        return pl.ds(pl.multiple_of(ebase + j * CHUNK, 8), CHUNK)

    def dslice(j):
        return pl.ds(pl.multiple_of(N_EDGES + ebase + j * CHUNK, 8), CHUNK)

    def start_idx(j, b):
        pltpu.async_copy(src_hbm.at[eslice(j)], sbufs[b], isems[b])
        pltpu.async_copy(src_hbm.at[dslice(j)], dbufs[b], isems[b])

    def wait_idx(j, b):
        pltpu.make_async_copy(src_hbm.at[eslice(j)], sbufs[b], isems[b]).wait()
        pltpu.make_async_copy(src_hbm.at[dslice(j)], dbufs[b], isems[b]).wait()

    def fire_gather(b):
        pltpu.async_copy(x_hbm.at[sbufs[b]], rows_at(b), gsems[b])

    def drain_gather(b):
        pltpu.make_async_copy(x_hbm.at[sbufs[b]], rows_at(b), gsems[b]).wait()

    def fire_scatter(b):
        for i in range(CHUNK // 16):
            cbufs[b][pl.ds(i * 16, 16)] = dbufs[b][pl.ds(i * 16, 16)]
        pltpu.async_copy(rows_at(b), he_sh.at[cbufs[b]], ssems[b], add=True)

    def drain_scatter(b):
        pltpu.make_async_copy(rows_at(b), he_sh.at[cbufs[b]], ssems[b]).wait()

    scope = jax.named_scope("edge_pipeline")
    scope.__enter__()

    def body(j, b, steady=True, f=True):
        if steady:
            drain_scatter(b)
        wait_idx(j, b)
        fire_gather(b)
        if steady or b >= 2:
            drain_gather((b - 2) % 4)
            fire_scatter((b - 2) % 4)
        if f:
            start_idx(j + 2, (b + 2) % 4)

    start_idx(0, 0)
    start_idx(1, 1)
    for j in range(4):
        body(j, j, steady=False)

    def _pipe(t, carry):
        j0 = 4 + 4 * t
        for u in range(4):
            body(j0 + u, u)
        return carry

    lax.fori_loop(0, (NCH - 4 - 5) // 4, _pipe, 0)

    for j in range(NCH - 5, NCH):
        body(j, j % 4, f=(j + 2 < NCH))
    drain_gather((NCH - 2) % 4)
    fire_scatter((NCH - 2) % 4)
    drain_gather((NCH - 1) % 4)
    fire_scatter((NCH - 1) % 4)
    for j in range(NCH - 4, NCH):
        drain_scatter(j % 4)
    scope.__exit__(None, None, None)

    with jax.named_scope("writeout"):
        plsc.subcore_barrier()
        pltpu.sync_copy(
            he_sh.at[pl.ds(sid * ROWS_PER_TILE, ROWS_PER_TILE)],
            he_hbm.at[cid, pl.ds(sid * ROWS_PER_TILE, ROWS_PER_TILE)])


_sc_segsum = pl.kernel(
    _sc_body,
    out_type=jax.ShapeDtypeStruct((NC, N_PAD, D), jnp.float32),
    mesh=plsc.VectorSubcoreMesh(core_axis_name="c", subcore_axis_name="s"),
    scratch_types=(
        [pltpu.VMEM((NSLOT * CHUNK, D), jnp.float32)]
        + [pltpu.VMEM((CHUNK,), jnp.int32) for _ in range(12)]
        + [pltpu.VMEM_SHARED((N_PAD, D), jnp.float32)]
        + [pltpu.SemaphoreType.DMA for _ in range(12)]
    ),
)


def _tc_body(x_ref, he_ref, w_ref, b_ref, o_ref):
    acc = x_ref[...] + math.pi * (he_ref[0] + he_ref[1])
    o_ref[...] = lax.dot_general(
        acc, w_ref[...], (((1,), (1,)), ((), ())),
        preferred_element_type=jnp.float32) + b_ref[...]


def _tc_linear(x, he, W, b2d):
    blk = 1000
    grid = N_NODES // blk
    return pl.pallas_call(
        _tc_body,
        grid=(grid,),
        in_specs=[
            pl.BlockSpec((blk, D), lambda i: (i, 0)),
            pl.BlockSpec((NC, blk, D), lambda i: (0, i, 0)),
            pl.BlockSpec((D, D), lambda i: (0, 0)),
            pl.BlockSpec((1, D), lambda i: (0, 0)),
        ],
        out_specs=pl.BlockSpec((blk, D), lambda i: (i, 0)),
        out_shape=jax.ShapeDtypeStruct((N_NODES, D), jnp.float32),
    )(x, he, W, b2d)


def kernel(x, edge_index, W, b):
    ei = jnp.reshape(edge_index, (2 * N_EDGES,))
    zeros = jnp.zeros((ROWS_PER_TILE, D), jnp.float32)
    he = _sc_segsum(x, ei, zeros)
    return _tc_linear(x, he, W, b.reshape(1, D))
